# baseline (speedup 1.0000x reference)
import functools
import math

import jax
import jax.numpy as jnp
from jax import lax
from jax.experimental import pallas as pl
from jax.experimental.pallas import tpu as pltpu
from jax.experimental.pallas import tpu_sc as plsc

D_MODEL = 64
SCALE = math.sqrt(D_MODEL)


@functools.partial(jax.jit, static_argnames=("N", "W", "D"))
def _emb_lookup(x, table, *, N, W, D):
    info = plsc.get_sparse_core_info()
    NC, NS, L = info.num_cores, info.num_subcores, info.num_lanes
    NW = NC * NS
    assert N % NW == 0
    slabs_per_w = N // NW
    S = 16
    n_chunks = slabs_per_w // S
    C = S * W
    assert D % L == 0

    mesh = plsc.VectorSubcoreMesh(core_axis_name="c", subcore_axis_name="s")

    @functools.partial(
        pl.kernel,
        mesh=mesh,
        out_type=jax.ShapeDtypeStruct((N, W, D), jnp.float32),
        scratch_types=[
            pltpu.VMEM((S, W), jnp.int32),
            pltpu.VMEM((S, W, D), jnp.float32),
            pltpu.SemaphoreType.DMA,
        ],
    )
    def k(x_hbm, table_hbm, out_hbm, xidx_v, rows_v, sem):
        wid = lax.axis_index("s") * NC + lax.axis_index("c")
        slab0 = wid * slabs_per_w

        for j in range(n_chunks):
            pltpu.sync_copy(x_hbm.at[pl.ds(slab0 + j * S, S)], xidx_v)

            def issue(r, carry):
                v0 = xidx_v[r, pl.ds(0, L)]
                v1 = xidx_v[r, pl.ds(L, L)]
                v2 = xidx_v[r, pl.ds(2 * L, L)]
                vt = xidx_v[r, pl.ds(W - L, L)]
                for t in range(W):
                    if t < 3 * L:
                        i = (v0, v1, v2)[t // L][t % L]
                    else:
                        i = vt[t - (W - L)]
                    pltpu.make_async_copy(
                        table_hbm.at[i], rows_v.at[r, t], sem
                    ).start()
                return carry

            lax.fori_loop(0, S, issue, 0)
            pltpu.make_async_copy(
                out_hbm.at[pl.ds(0, S)], rows_v, sem
            ).wait()

            def scale_slab(s, carry):
                def scale_row(t, carry2):
                    for g in range(D // L):
                        sl = (s, t, pl.ds(g * L, L))
                        rows_v[sl] = rows_v[sl] * SCALE
                    return carry2

                return lax.fori_loop(0, W, scale_row, carry)

            lax.fori_loop(0, S, scale_slab, 0)

            for i in range(S):
                pltpu.sync_copy(
                    rows_v.at[i], out_hbm.at[slab0 + j * S + i]
                )

    return k(x, table)


def kernel(x, word_emb_weight):
    N, W = x.shape
    D = word_emb_weight.shape[1]
    return _emb_lookup(x, word_emb_weight, N=N, W=W, D=D)
